# Initial kernel scaffold; baseline (speedup 1.0000x reference)
import jax
import jax.numpy as jnp
from jax.experimental import pallas as pl


def kernel(users, items, gum_temp, hard, edge_index, edge_weight, user_emb, item_emb, g1_W1, g1_b1, g1_W2, g1_b2, g2_W1, g2_b1, g2_W2, g2_b2):
    raise NotImplementedError("write your pallas kernel here")


# speedup vs baseline: 10.5140x; 10.5140x over previous
import functools

import jax
import jax.numpy as jnp
from jax import lax
from jax.experimental import pallas as pl
from jax.experimental.pallas import tpu as pltpu
from jax.experimental.pallas import tpu_sc as plsc

_DIVISION_NOISE = 3.0
_NC = 2
_NS = 16
_L = 16
_K = 1024
_GBLK = 512


def _ceil_to(x, m):
    return ((x + m - 1) // m) * m


def _spmm_body(np_rows, ept, src2d, dst2d, w_hbm, x0, x1, out0, out1,
               idx_v, dst_v, w_v, rows_v, acc, sem):
    c = lax.axis_index("c")
    s = lax.axis_index("s")
    rows_per_tile = np_rows // _NS
    zrows = 512
    zc = rows_per_tile // zrows

    def _zrow(i, carry):
        rows_v[i, :] = jnp.zeros((_L,), jnp.float32)
        return carry
    lax.fori_loop(0, zrows, _zrow, 0)
    row0 = s * rows_per_tile
    for r in range(zc):
        pltpu.sync_copy(rows_v.at[pl.ds(0, zrows)],
                        acc.at[pl.ds(row0 + r * zrows, zrows)])
    plsc.subcore_barrier()

    nchunks = ept // _K
    iota = lax.iota(jnp.int32, _L)

    def _chunk(i, carry):
        e0 = pl.multiple_of(s * ept + i * _K, 1024)
        r128 = pl.multiple_of((s * ept + i * _K) // 128, 8)
        pltpu.sync_copy(src2d.at[pl.ds(r128, _K // 128)], idx_v)
        pltpu.sync_copy(dst2d.at[pl.ds(r128, _K // 128)], dst_v)
        pltpu.sync_copy(w_hbm.at[pl.ds(e0, _K)], w_v)

        @pl.when(c == 0)
        def _():
            cps = [pltpu.async_copy(x0.at[idx_v.at[j]],
                                    rows_v.at[pl.ds(j * 128, 128)], sem)
                   for j in range(_K // 128)]
            for cp in cps:
                cp.wait()

        @pl.when(c == 1)
        def _():
            cps = [pltpu.async_copy(x1.at[idx_v.at[j]],
                                    rows_v.at[pl.ds(j * 128, 128)], sem)
                   for j in range(_K // 128)]
            for cp in cps:
                cp.wait()

        def _grp(g, carry2):
            wv = w_v[pl.ds(g * _L, _L)]
            e0g = g * _L
            for k in range(_L):
                b = wv[jnp.full((_L,), k, jnp.int32)]
                rows_v[e0g + k, :] = rows_v[e0g + k, :] * b
            return carry2
        lax.fori_loop(0, _K // _L, _grp, 0)

        for j in range(_K // 128):
            pltpu.sync_copy(rows_v.at[pl.ds(j * 128, 128)],
                            acc.at[dst_v.at[j]], add=True)
        return carry
    lax.fori_loop(0, nchunks, _chunk, 0)
    plsc.subcore_barrier()

    @pl.when(c == 0)
    def _():
        pltpu.sync_copy(acc.at[pl.ds(row0, rows_per_tile)],
                        out0.at[pl.ds(row0, rows_per_tile)])

    @pl.when(c == 1)
    def _():
        pltpu.sync_copy(acc.at[pl.ds(row0, rows_per_tile)],
                        out1.at[pl.ds(row0, rows_per_tile)])


def _spmm(src2d, dst2d, w_hbm, x0, x1, np_rows):
    ep = w_hbm.shape[0]
    ept = ep // _NS
    mesh = plsc.VectorSubcoreMesh(core_axis_name="c", subcore_axis_name="s")
    body = functools.partial(_spmm_body, np_rows, ept)
    f = pl.kernel(
        body,
        out_type=(jax.ShapeDtypeStruct((np_rows, _L), jnp.float32),
                  jax.ShapeDtypeStruct((np_rows, _L), jnp.float32)),
        mesh=mesh,
        compiler_params=pltpu.CompilerParams(use_tc_tiling_on_sc=False),
        scratch_types=[
            pltpu.VMEM((_K // 128, 128), jnp.int32),
            pltpu.VMEM((_K // 128, 128), jnp.int32),
            pltpu.VMEM((_K,), jnp.float32),
            pltpu.VMEM((_K, _L), jnp.float32),
            pltpu.VMEM_SHARED((np_rows, _L), jnp.float32),
            pltpu.SemaphoreType.DMA,
        ],
    )
    return f(src2d, dst2d, w_hbm, x0, x1)


def _gate_body(n_nodes, xl0, xl1, xn0, xn1, ka, kb, kc, kd, b1r, k0, k1,
               nd0, nd1, e0, e1, cnt):
    pid = pl.program_id(0)
    x_l0 = xl0[...]
    x_l1 = xl1[...]
    n0 = xn0[...]
    n1 = xn1[...]
    n0 = jnp.where(n0 > 0, n0, jnp.exp(jnp.minimum(n0, 0.0)) - 1.0)
    n1 = jnp.where(n1 > 0, n1, jnp.exp(jnp.minimum(n1, 0.0)) - 1.0)
    bf = jnp.bfloat16
    h = (jnp.dot(x_l0.astype(bf), ka[...], preferred_element_type=jnp.float32)
         + jnp.dot(x_l1.astype(bf), kb[...], preferred_element_type=jnp.float32)
         + jnp.dot(n0.astype(bf), kc[...], preferred_element_type=jnp.float32)
         + jnp.dot(n1.astype(bf), kd[...], preferred_element_type=jnp.float32))
    h = jnp.maximum(h + b1r[...], 0.0).astype(bf)
    t0 = jnp.dot(h, k0[...], preferred_element_type=jnp.float32) + nd0[...]
    t1 = jnp.dot(h, k1[...], preferred_element_type=jnp.float32) + nd1[...]
    mask = t0 >= t1
    e0[...] = jnp.where(mask, x_l0, n0)
    e1[...] = jnp.where(mask, x_l1, n1)
    rows = jax.lax.broadcasted_iota(jnp.int32, (_GBLK, 128), 0)
    lanes = jax.lax.broadcasted_iota(jnp.int32, (_GBLK, 128), 1)
    nid = (pid * _GBLK + rows) * 8 + lanes // _L
    valid = jnp.logical_and(mask, nid < n_nodes)
    partial = valid.astype(jnp.int32).reshape(_GBLK // 8, 8, 128).sum(axis=0)

    @pl.when(pid == 0)
    def _():
        cnt[...] = jnp.zeros((8, 128), jnp.int32)
    cnt[...] += partial


def _gate(xl0, xl1, xn0, xn1, w1, b1, w2, b2, nd0_rep, nd1_rep, n_nodes):
    np_rows = xl0.shape[0]
    mp = np_rows // 8
    grid = mp // _GBLK
    bf = jnp.bfloat16
    eye8 = jnp.eye(8, dtype=jnp.float32)
    ks = [jnp.kron(eye8, w1[j * _L:(j + 1) * _L, :]).astype(bf)
          for j in range(4)]
    ones16 = jnp.ones((_L,), jnp.float32)
    k0 = jnp.kron(eye8, jnp.outer(w2[:, 0], ones16)).astype(bf)
    k1 = jnp.kron(eye8, jnp.outer(w2[:, 1], ones16)).astype(bf)
    b1r = jnp.tile(b1, 8)[None, :]

    xspec = pl.BlockSpec((_GBLK, 128), lambda i: (i, 0))
    wspec = pl.BlockSpec((128, 128), lambda i: (0, 0))
    bspec = pl.BlockSpec((1, 128), lambda i: (0, 0))
    cspec = pl.BlockSpec((8, 128), lambda i: (0, 0))

    packed = lambda a: a.reshape(mp, 128)
    out = pl.pallas_call(
        functools.partial(_gate_body, n_nodes),
        grid=(grid,),
        in_specs=[xspec, xspec, xspec, xspec, wspec, wspec, wspec, wspec,
                  bspec, wspec, wspec, xspec, xspec],
        out_specs=[xspec, xspec, cspec],
        out_shape=[jax.ShapeDtypeStruct((mp, 128), jnp.float32),
                   jax.ShapeDtypeStruct((mp, 128), jnp.float32),
                   jax.ShapeDtypeStruct((8, 128), jnp.int32)],
    )(packed(xl0), packed(xl1), packed(xn0), packed(xn1),
      ks[0], ks[1], ks[2], ks[3], b1r, k0, k1, nd0_rep, nd1_rep)
    emb0 = out[0].reshape(np_rows, _L)
    emb1 = out[1].reshape(np_rows, _L)
    lc = jnp.sum(out[2]) // _L
    return emb0, emb1, lc


def _gamma_body(nu, b_per_w, users_hbm, items_hbm, a00, a01, a10, a11, a20,
                a21, a30, a31, a40, a41, prods_hbm, uidx, iidx, bufs, prod_v,
                sem):
    c = lax.axis_index("c")
    s = lax.axis_index("s")
    wid = s * _NC + c
    base = pl.multiple_of(wid * b_per_w, b_per_w)
    pltpu.sync_copy(users_hbm.at[pl.ds(base, b_per_w)], uidx)
    pltpu.sync_copy(items_hbm.at[pl.ds(base, b_per_w)], iidx)

    def _shift(g, carry):
        iidx[pl.ds(g * _L, _L)] = iidx[pl.ds(g * _L, _L)] + nu
        return carry
    lax.fori_loop(0, b_per_w // _L, _shift, 0)

    arrs = [a00, a01, a10, a11, a20, a21, a30, a31, a40, a41]
    cps = []
    for k, arr in enumerate(arrs):
        cps.append(pltpu.async_copy(arr.at[uidx], bufs.at[2 * k], sem))
        cps.append(pltpu.async_copy(arr.at[iidx], bufs.at[2 * k + 1], sem))
    for cp in cps:
        cp.wait()

    def _pair(p, carry):
        us0 = (bufs[0, p, :] + bufs[4, p, :] + bufs[8, p, :]
               + bufs[12, p, :] + bufs[16, p, :])
        is0 = (bufs[1, p, :] + bufs[5, p, :] + bufs[9, p, :]
               + bufs[13, p, :] + bufs[17, p, :])
        us1 = (bufs[2, p, :] + bufs[6, p, :] + bufs[10, p, :]
               + bufs[14, p, :] + bufs[18, p, :])
        is1 = (bufs[3, p, :] + bufs[7, p, :] + bufs[11, p, :]
               + bufs[15, p, :] + bufs[19, p, :])
        prod_v[p, :] = (us0 * is0 + us1 * is1) * (1.0 / 25.0)
        return carry
    lax.fori_loop(0, b_per_w, _pair, 0)
    pltpu.sync_copy(prod_v, prods_hbm.at[pl.ds(base, b_per_w)])


def _gamma_fin_body(x_ref, o_ref):
    x = x_ref[...]
    m = x.shape[0]
    s = x.reshape(m, 8, _L).sum(axis=2)
    o_ref[...] = s.reshape(m // 16, 128)


def _gamma(users, items, arrs, nu):
    b = users.shape[0]
    b_per_w = b // (_NC * _NS)
    mesh = plsc.VectorSubcoreMesh(core_axis_name="c", subcore_axis_name="s")
    f = pl.kernel(
        functools.partial(_gamma_body, nu, b_per_w),
        out_type=jax.ShapeDtypeStruct((b, _L), jnp.float32),
        mesh=mesh,
        compiler_params=pltpu.CompilerParams(use_tc_tiling_on_sc=False),
        scratch_types=[
            pltpu.VMEM((b_per_w,), jnp.int32),
            pltpu.VMEM((b_per_w,), jnp.int32),
            pltpu.VMEM((20, b_per_w, _L), jnp.float32),
            pltpu.VMEM((b_per_w, _L), jnp.float32),
            pltpu.SemaphoreType.DMA,
        ],
    )
    prods = f(users, items, *arrs)
    gamma2d = pl.pallas_call(
        _gamma_fin_body,
        out_shape=jax.ShapeDtypeStruct((b // 128, 128), jnp.float32),
    )(prods.reshape(b // 8, 128))
    return gamma2d.reshape(b)


def _gumbel_cols(key, n, b2):
    u = jax.random.uniform(key, (n, 2), minval=1e-20, maxval=1.0)
    g = -jnp.log(-jnp.log(u))
    return g[:, 0] / _DIVISION_NOISE + b2[0], g[:, 1] / _DIVISION_NOISE + b2[1]


def kernel(users, items, gum_temp, hard, edge_index, edge_weight, user_emb,
           item_emb, g1_W1, g1_b1, g1_W2, g1_b2, g2_W1, g2_b1, g2_W2, g2_b2):
    nu = user_emb.shape[0]
    n = nu + item_emb.shape[0]
    e = edge_index.shape[1]
    np_rows = _ceil_to(n, 8 * _GBLK * 2)

    ep = _ceil_to(e, _NS * _K)
    pad = ep - e
    fill = (jnp.arange(pad, dtype=jnp.int32) * 97) % n
    src = jnp.concatenate([edge_index[0].astype(jnp.int32), fill])
    dst = jnp.concatenate([edge_index[1].astype(jnp.int32), fill])
    w = jnp.concatenate([edge_weight, jnp.zeros((pad,), jnp.float32)])
    src2d = src.reshape(ep // 128, 128)
    dst2d = dst.reshape(ep // 128, 128)

    all0 = jnp.concatenate([user_emb, item_emb], axis=0)
    a0 = (all0[:, :_L], all0[:, _L:])

    e1 = _spmm(src2d, dst2d, w, a0[0], a0[1], np_rows)
    e2 = _spmm(src2d, dst2d, w, e1[0], e1[1], np_rows)
    e3 = _spmm(src2d, dst2d, w, e2[0], e2[1], np_rows)

    na1, nb1 = _gumbel_cols(jax.random.fold_in(jax.random.key(1), 1), n, g1_b2)
    na2, nb2 = _gumbel_cols(jax.random.fold_in(jax.random.key(1), 2), n, g2_b2)
    mp = np_rows // 8

    def _rep(ndv):
        r = jnp.repeat(ndv, _L)
        return jnp.pad(r, (0, (np_rows - n) * _L)).reshape(mp, 128)

    m1_0, m1_1, lc3 = _gate(e3[0], e3[1], e1[0], e1[1], g1_W1, g1_b1, g1_W2,
                            g1_b2, _rep(na1), _rep(nb1), n)
    e4 = _spmm(src2d, dst2d, w, m1_0, m1_1, np_rows)
    m2_0, m2_1, lc4 = _gate(e4[0], e4[1], e4[0], e4[1], g2_W1, g2_b1, g2_W2,
                            g2_b2, _rep(na2), _rep(nb2), n)

    arrs = [a0[0], a0[1], e1[0], e1[1], e2[0], e2[1], m1_0, m1_1, m2_0, m2_1]
    gamma = _gamma(users.astype(jnp.int32), items.astype(jnp.int32), arrs, nu)

    cat = lambda p0, p1: jnp.concatenate([p0[:n], p1[:n]], axis=1)
    embs = jnp.stack([all0, cat(e1[0], e1[1]), cat(e2[0], e2[1]),
                      cat(m1_0, m1_1), cat(m2_0, m2_1)], axis=1)
    nc3 = n - lc3
    nc4 = n - lc4
    return (gamma, lc3, nc3, lc4, nc4, embs)


# speedup vs baseline: 13.5437x; 1.2882x over previous
import functools

import jax
import jax.numpy as jnp
from jax import lax
from jax.experimental import pallas as pl
from jax.experimental.pallas import tpu as pltpu
from jax.experimental.pallas import tpu_sc as plsc

_DIVISION_NOISE = 3.0
_NC = 2
_NS = 16
_L = 16
_K = 512
_GBLK = 512


def _ceil_to(x, m):
    return ((x + m - 1) // m) * m


_KJ = _K // 128


def _spmm_body(np_rows, nchunks, comb, w_hbm, x0, x1, out0, out1,
               comb_v, w_v, rows_v, acc, semg0, semg1, seme0, seme1):
    c = lax.axis_index("c")
    s = lax.axis_index("s")
    semg = (semg0, semg1)
    seme = (seme0, seme1)
    rows_per_tile = np_rows // _NS

    def _fire_loads(j, bb):
        pltpu.async_copy(comb.at[s * nchunks + j], comb_v.at[bb], seme[bb])
        e0 = pl.multiple_of((s * nchunks + j) * _K, _K)
        pltpu.async_copy(w_hbm.at[pl.ds(e0, _K)], w_v.at[bb], seme[bb])

    def _wait_loads(j, bb):
        pltpu.make_async_copy(comb.at[s * nchunks + j], comb_v.at[bb],
                              seme[bb]).wait()
        e0 = pl.multiple_of((s * nchunks + j) * _K, _K)
        pltpu.make_async_copy(w_hbm.at[pl.ds(e0, _K)], w_v.at[bb],
                              seme[bb]).wait()

    def _fire_gathers(bb):
        @pl.when(c == 0)
        def _():
            for jj in range(_KJ):
                pltpu.async_copy(x0.at[comb_v.at[bb, jj]],
                                 rows_v.at[bb, pl.ds(jj * 128, 128)],
                                 semg[bb])

        @pl.when(c == 1)
        def _():
            for jj in range(_KJ):
                pltpu.async_copy(x1.at[comb_v.at[bb, jj]],
                                 rows_v.at[bb, pl.ds(jj * 128, 128)],
                                 semg[bb])

    def _wait_gathers(bb):
        for jj in range(_KJ):
            pltpu.make_async_copy(x0.at[comb_v.at[bb, jj]],
                                  rows_v.at[bb, pl.ds(jj * 128, 128)],
                                  semg[bb]).wait()

    def _scale(bb):
        def _grp(g, carry2):
            wv = w_v[bb, pl.ds(g * _L, _L)]
            e0g = g * _L
            for k in range(_L):
                b = wv[jnp.full((_L,), k, jnp.int32)]
                rows_v[bb, e0g + k, :] = rows_v[bb, e0g + k, :] * b
            return carry2
        lax.fori_loop(0, _K // _L, _grp, 0)

    def _scatter(bb):
        for jj in range(_KJ):
            pltpu.sync_copy(rows_v.at[bb, pl.ds(jj * 128, 128)],
                            acc.at[comb_v.at[bb, _KJ + jj]], add=True)

    def _zrow(i, carry):
        rows_v[0, i, :] = jnp.zeros((_L,), jnp.float32)
        return carry
    lax.fori_loop(0, 512, _zrow, 0)
    row0 = s * rows_per_tile
    for r in range(rows_per_tile // 512):
        pltpu.sync_copy(rows_v.at[0, pl.ds(0, 512)],
                        acc.at[pl.ds(row0 + r * 512, 512)])
    plsc.subcore_barrier()

    _fire_loads(0, 0)
    _wait_loads(0, 0)
    _fire_gathers(0)

    def _pair(p, carry):
        i0 = 2 * p
        _wait_gathers(0)
        _fire_loads(i0 + 1, 1)
        _scale(0)
        _wait_loads(i0 + 1, 1)
        _fire_gathers(1)
        _scatter(0)
        _wait_gathers(1)

        @pl.when(i0 + 2 < nchunks)
        def _():
            _fire_loads(i0 + 2, 0)
        _scale(1)

        @pl.when(i0 + 2 < nchunks)
        def _():
            _wait_loads(i0 + 2, 0)
            _fire_gathers(0)
        _scatter(1)
        return carry
    lax.fori_loop(0, nchunks // 2, _pair, 0)
    plsc.subcore_barrier()

    @pl.when(c == 0)
    def _():
        pltpu.sync_copy(acc.at[pl.ds(row0, rows_per_tile)],
                        out0.at[pl.ds(row0, rows_per_tile)])

    @pl.when(c == 1)
    def _():
        pltpu.sync_copy(acc.at[pl.ds(row0, rows_per_tile)],
                        out1.at[pl.ds(row0, rows_per_tile)])


def _spmm(comb, w_hbm, x0, x1, np_rows):
    nchunks = comb.shape[0] // _NS
    mesh = plsc.VectorSubcoreMesh(core_axis_name="c", subcore_axis_name="s")
    body = functools.partial(_spmm_body, np_rows, nchunks)
    f = pl.kernel(
        body,
        out_type=(jax.ShapeDtypeStruct((np_rows, _L), jnp.float32),
                  jax.ShapeDtypeStruct((np_rows, _L), jnp.float32)),
        mesh=mesh,
        compiler_params=pltpu.CompilerParams(use_tc_tiling_on_sc=False),
        scratch_types=[
            pltpu.VMEM((2, 2 * _KJ, 128), jnp.int32),
            pltpu.VMEM((2, _K), jnp.float32),
            pltpu.VMEM((2, _K, _L), jnp.float32),
            pltpu.VMEM_SHARED((np_rows, _L), jnp.float32),
            pltpu.SemaphoreType.DMA,
            pltpu.SemaphoreType.DMA,
            pltpu.SemaphoreType.DMA,
            pltpu.SemaphoreType.DMA,
        ],
    )
    return f(comb, w_hbm, x0, x1)


def _gate_body(n_nodes, xl0, xl1, xn0, xn1, ka, kb, kc, kd, b1r, k0, k1,
               nd0, nd1, e0, e1, cnt):
    pid = pl.program_id(0)
    x_l0 = xl0[...]
    x_l1 = xl1[...]
    n0 = xn0[...]
    n1 = xn1[...]
    n0 = jnp.where(n0 > 0, n0, jnp.exp(jnp.minimum(n0, 0.0)) - 1.0)
    n1 = jnp.where(n1 > 0, n1, jnp.exp(jnp.minimum(n1, 0.0)) - 1.0)
    bf = jnp.bfloat16
    h = (jnp.dot(x_l0.astype(bf), ka[...], preferred_element_type=jnp.float32)
         + jnp.dot(x_l1.astype(bf), kb[...], preferred_element_type=jnp.float32)
         + jnp.dot(n0.astype(bf), kc[...], preferred_element_type=jnp.float32)
         + jnp.dot(n1.astype(bf), kd[...], preferred_element_type=jnp.float32))
    h = jnp.maximum(h + b1r[...], 0.0).astype(bf)
    t0 = jnp.dot(h, k0[...], preferred_element_type=jnp.float32) + nd0[...]
    t1 = jnp.dot(h, k1[...], preferred_element_type=jnp.float32) + nd1[...]
    mask = t0 >= t1
    e0[...] = jnp.where(mask, x_l0, n0)
    e1[...] = jnp.where(mask, x_l1, n1)
    rows = jax.lax.broadcasted_iota(jnp.int32, (_GBLK, 128), 0)
    lanes = jax.lax.broadcasted_iota(jnp.int32, (_GBLK, 128), 1)
    nid = (pid * _GBLK + rows) * 8 + lanes // _L
    valid = jnp.logical_and(mask, nid < n_nodes)
    partial = valid.astype(jnp.int32).reshape(_GBLK // 8, 8, 128).sum(axis=0)

    @pl.when(pid == 0)
    def _():
        cnt[...] = jnp.zeros((8, 128), jnp.int32)
    cnt[...] += partial


def _gate(xl0, xl1, xn0, xn1, w1, b1, w2, b2, nd0_rep, nd1_rep, n_nodes):
    np_rows = xl0.shape[0]
    mp = np_rows // 8
    grid = mp // _GBLK
    bf = jnp.bfloat16
    eye8 = jnp.eye(8, dtype=jnp.float32)
    ks = [jnp.kron(eye8, w1[j * _L:(j + 1) * _L, :]).astype(bf)
          for j in range(4)]
    ones16 = jnp.ones((_L,), jnp.float32)
    k0 = jnp.kron(eye8, jnp.outer(w2[:, 0], ones16)).astype(bf)
    k1 = jnp.kron(eye8, jnp.outer(w2[:, 1], ones16)).astype(bf)
    b1r = jnp.tile(b1, 8)[None, :]

    xspec = pl.BlockSpec((_GBLK, 128), lambda i: (i, 0))
    wspec = pl.BlockSpec((128, 128), lambda i: (0, 0))
    bspec = pl.BlockSpec((1, 128), lambda i: (0, 0))
    cspec = pl.BlockSpec((8, 128), lambda i: (0, 0))

    packed = lambda a: a.reshape(mp, 128)
    out = pl.pallas_call(
        functools.partial(_gate_body, n_nodes),
        grid=(grid,),
        in_specs=[xspec, xspec, xspec, xspec, wspec, wspec, wspec, wspec,
                  bspec, wspec, wspec, xspec, xspec],
        out_specs=[xspec, xspec, cspec],
        out_shape=[jax.ShapeDtypeStruct((mp, 128), jnp.float32),
                   jax.ShapeDtypeStruct((mp, 128), jnp.float32),
                   jax.ShapeDtypeStruct((8, 128), jnp.int32)],
    )(packed(xl0), packed(xl1), packed(xn0), packed(xn1),
      ks[0], ks[1], ks[2], ks[3], b1r, k0, k1, nd0_rep, nd1_rep)
    emb0 = out[0].reshape(np_rows, _L)
    emb1 = out[1].reshape(np_rows, _L)
    lc = jnp.sum(out[2]) // _L
    return emb0, emb1, lc


def _gamma_body(nu, b_per_w, users_hbm, items_hbm, a00, a01, a10, a11, a20,
                a21, a30, a31, a40, a41, prods_hbm, uidx, iidx, bufs, prod_v,
                sem):
    c = lax.axis_index("c")
    s = lax.axis_index("s")
    wid = s * _NC + c
    base = pl.multiple_of(wid * b_per_w, b_per_w)
    pltpu.sync_copy(users_hbm.at[pl.ds(base, b_per_w)], uidx)
    pltpu.sync_copy(items_hbm.at[pl.ds(base, b_per_w)], iidx)

    def _shift(g, carry):
        iidx[pl.ds(g * _L, _L)] = iidx[pl.ds(g * _L, _L)] + nu
        return carry
    lax.fori_loop(0, b_per_w // _L, _shift, 0)

    arrs = [a00, a01, a10, a11, a20, a21, a30, a31, a40, a41]
    cps = []
    for k, arr in enumerate(arrs):
        cps.append(pltpu.async_copy(arr.at[uidx], bufs.at[2 * k], sem))
        cps.append(pltpu.async_copy(arr.at[iidx], bufs.at[2 * k + 1], sem))
    for cp in cps:
        cp.wait()

    def _pair(p, carry):
        us0 = (bufs[0, p, :] + bufs[4, p, :] + bufs[8, p, :]
               + bufs[12, p, :] + bufs[16, p, :])
        is0 = (bufs[1, p, :] + bufs[5, p, :] + bufs[9, p, :]
               + bufs[13, p, :] + bufs[17, p, :])
        us1 = (bufs[2, p, :] + bufs[6, p, :] + bufs[10, p, :]
               + bufs[14, p, :] + bufs[18, p, :])
        is1 = (bufs[3, p, :] + bufs[7, p, :] + bufs[11, p, :]
               + bufs[15, p, :] + bufs[19, p, :])
        prod_v[p, :] = (us0 * is0 + us1 * is1) * (1.0 / 25.0)
        return carry
    lax.fori_loop(0, b_per_w, _pair, 0)
    pltpu.sync_copy(prod_v, prods_hbm.at[pl.ds(base, b_per_w)])


def _gamma_fin_body(x_ref, o_ref):
    x = x_ref[...]
    m = x.shape[0]
    s = x.reshape(m, 8, _L).sum(axis=2)
    o_ref[...] = s.reshape(m // 16, 128)


def _gamma(users, items, arrs, nu):
    b = users.shape[0]
    b_per_w = b // (_NC * _NS)
    mesh = plsc.VectorSubcoreMesh(core_axis_name="c", subcore_axis_name="s")
    f = pl.kernel(
        functools.partial(_gamma_body, nu, b_per_w),
        out_type=jax.ShapeDtypeStruct((b, _L), jnp.float32),
        mesh=mesh,
        compiler_params=pltpu.CompilerParams(use_tc_tiling_on_sc=False),
        scratch_types=[
            pltpu.VMEM((b_per_w,), jnp.int32),
            pltpu.VMEM((b_per_w,), jnp.int32),
            pltpu.VMEM((20, b_per_w, _L), jnp.float32),
            pltpu.VMEM((b_per_w, _L), jnp.float32),
            pltpu.SemaphoreType.DMA,
        ],
    )
    prods = f(users, items, *arrs)
    gamma2d = pl.pallas_call(
        _gamma_fin_body,
        out_shape=jax.ShapeDtypeStruct((b // 128, 128), jnp.float32),
    )(prods.reshape(b // 8, 128))
    return gamma2d.reshape(b)


def _gumbel_cols(key, n, b2):
    u = jax.random.uniform(key, (n, 2), minval=1e-20, maxval=1.0)
    g = -jnp.log(-jnp.log(u))
    return g[:, 0] / _DIVISION_NOISE + b2[0], g[:, 1] / _DIVISION_NOISE + b2[1]


def kernel(users, items, gum_temp, hard, edge_index, edge_weight, user_emb,
           item_emb, g1_W1, g1_b1, g1_W2, g1_b2, g2_W1, g2_b1, g2_W2, g2_b2):
    nu = user_emb.shape[0]
    n = nu + item_emb.shape[0]
    e = edge_index.shape[1]
    np_rows = _ceil_to(n, 8 * _GBLK * 2)

    ep = _ceil_to(e, _NS * _K)
    pad = ep - e
    fill = (jnp.arange(pad, dtype=jnp.int32) * 97) % n
    src = jnp.concatenate([edge_index[0].astype(jnp.int32), fill])
    dst = jnp.concatenate([edge_index[1].astype(jnp.int32), fill])
    w = jnp.concatenate([edge_weight, jnp.zeros((pad,), jnp.float32)])
    comb = jnp.concatenate(
        [src.reshape(-1, _KJ, 128), dst.reshape(-1, _KJ, 128)],
        axis=1)

    all0 = jnp.concatenate([user_emb, item_emb], axis=0)
    a0 = (all0[:, :_L], all0[:, _L:])

    e1 = _spmm(comb, w, a0[0], a0[1], np_rows)
    e2 = _spmm(comb, w, e1[0], e1[1], np_rows)
    e3 = _spmm(comb, w, e2[0], e2[1], np_rows)

    na1, nb1 = _gumbel_cols(jax.random.fold_in(jax.random.key(1), 1), n, g1_b2)
    na2, nb2 = _gumbel_cols(jax.random.fold_in(jax.random.key(1), 2), n, g2_b2)
    mp = np_rows // 8

    def _rep(ndv):
        r = jnp.repeat(ndv, _L)
        return jnp.pad(r, (0, (np_rows - n) * _L)).reshape(mp, 128)

    m1_0, m1_1, lc3 = _gate(e3[0], e3[1], e1[0], e1[1], g1_W1, g1_b1, g1_W2,
                            g1_b2, _rep(na1), _rep(nb1), n)
    e4 = _spmm(comb, w, m1_0, m1_1, np_rows)
    m2_0, m2_1, lc4 = _gate(e4[0], e4[1], e4[0], e4[1], g2_W1, g2_b1, g2_W2,
                            g2_b2, _rep(na2), _rep(nb2), n)

    arrs = [a0[0], a0[1], e1[0], e1[1], e2[0], e2[1], m1_0, m1_1, m2_0, m2_1]
    gamma = _gamma(users.astype(jnp.int32), items.astype(jnp.int32), arrs, nu)

    cat = lambda p0, p1: jnp.concatenate([p0[:n], p1[:n]], axis=1)
    embs = jnp.stack([all0, cat(e1[0], e1[1]), cat(e2[0], e2[1]),
                      cat(m1_0, m1_1), cat(m2_0, m2_1)], axis=1)
    nc3 = n - lc3
    nc4 = n - lc4
    return (gamma, lc3, nc3, lc4, nc4, embs)
